# Initial kernel scaffold; baseline (speedup 1.0000x reference)
import jax
import jax.numpy as jnp
from jax.experimental import pallas as pl


def kernel(x, y, E0, E1, E2, E3, E4):
    raise NotImplementedError("write your pallas kernel here")


# speedup vs baseline: 3.2616x; 3.2616x over previous
import functools

import jax
import jax.numpy as jnp
from jax import lax
from jax.experimental import pallas as pl
from jax.experimental.pallas import tpu as pltpu
from jax.experimental.pallas import tpu_sc as plsc

VOCAB_SIZES = (6, 7, 12, 7, 96)
EMB_DIMS = (3, 4, 6, 4, 50)
N_ROWS = 16384
X_COLS = 64
OUT_COLS = X_COLS + sum(EMB_DIMS)

NUM_CORES = 2
NUM_SUBCORES = 16
NUM_WORKERS = NUM_CORES * NUM_SUBCORES
ROWS_PER_W = N_ROWS // NUM_WORKERS
LANES = 16
CHUNKS = ROWS_PER_W // LANES

_TAB_BASE = []
_acc = 0
for _v, _d in zip(VOCAB_SIZES, EMB_DIMS):
    _TAB_BASE.append(_acc)
    _acc += _v * _d
TAB_WORDS = _acc
TAB_PAD = (TAB_WORDS + 15) // 16 * 16

_COL_OFF = []
_acc = X_COLS
for _d in EMB_DIMS:
    _COL_OFF.append(_acc)
    _acc += _d


def _body(x_hbm, y_hbm, tab_hbm, out_hbm, x_v, y_v, tab_v, out_v):
    wid = lax.axis_index("s") * NUM_CORES + lax.axis_index("c")
    base = wid * ROWS_PER_W

    pltpu.sync_copy(x_hbm.at[pl.ds(base * X_COLS, ROWS_PER_W * X_COLS)], x_v)
    pltpu.sync_copy(y_hbm.at[pl.ds(base * 5, ROWS_PER_W * 5)], y_v)
    pltpu.sync_copy(tab_hbm, tab_v)

    iota = lax.broadcasted_iota(jnp.int32, (LANES,), 0)

    def chunk(c, carry):
        rows = c * LANES + iota
        rx = rows * X_COLS
        ro = rows * OUT_COLS
        for j in range(X_COLS):
            val = plsc.load_gather(x_v, [rx + j])
            plsc.store_scatter(out_v, [ro + j], val)
        ry = rows * 5
        for i in range(5):
            yi = plsc.load_gather(y_v, [ry + i])
            addr = yi * EMB_DIMS[i] + _TAB_BASE[i]
            for cc in range(EMB_DIMS[i]):
                val = plsc.load_gather(tab_v, [addr + cc])
                plsc.store_scatter(out_v, [ro + (_COL_OFF[i] + cc)], val)
        return carry

    lax.fori_loop(0, CHUNKS, chunk, 0)
    pltpu.sync_copy(out_v, out_hbm.at[pl.ds(base * OUT_COLS,
                                            ROWS_PER_W * OUT_COLS)])


_feature_call = functools.partial(
    pl.kernel,
    out_type=jax.ShapeDtypeStruct((N_ROWS * OUT_COLS,), jnp.float32),
    mesh=plsc.VectorSubcoreMesh(core_axis_name="c", subcore_axis_name="s"),
    compiler_params=pltpu.CompilerParams(needs_layout_passes=False),
    scratch_types=[
        pltpu.VMEM((ROWS_PER_W * X_COLS,), jnp.float32),
        pltpu.VMEM((ROWS_PER_W * 5,), jnp.int32),
        pltpu.VMEM((TAB_PAD,), jnp.float32),
        pltpu.VMEM((ROWS_PER_W * OUT_COLS,), jnp.float32),
    ],
)(_body)


def kernel(x, y, E0, E1, E2, E3, E4):
    tab = jnp.concatenate(
        [jnp.reshape(t, (-1,)) for t in (E0, E1, E2, E3, E4)]
        + [jnp.zeros((TAB_PAD - TAB_WORDS,), jnp.float32)])
    out_flat = _feature_call(jnp.reshape(x, (-1,)), jnp.reshape(y, (-1,)), tab)
    return jnp.reshape(out_flat, (N_ROWS, OUT_COLS))
